# baseline (speedup 1.0000x reference)
import functools

import jax
import jax.numpy as jnp
from jax import lax
from jax.experimental import pallas as pl
from jax.experimental.pallas import tpu as pltpu
from jax.experimental.pallas import tpu_sc as plsc

N = 10000
E = 320000
DF = 128
DE = 16
TE = 3
TN = 2

NW = 32
CHUNK = 128
NCHUNK = 80
PW = CHUNK * NCHUNK
EP = NW * PW
ROWS = EP * DE // 128
ROWS_REAL = E * DE // 128

_HIGH = lax.Precision.HIGHEST


def _dot(a, b):
    return lax.dot_general(a, b, (((1,), (0,)), ((), ())),
                           precision=_HIGH, preferred_element_type=jnp.float32)


def _proj_body(nf_ref, w_ref, o_ref):
    o_ref[...] = _dot(nf_ref[...], w_ref[...])


def _proj(nf, wpad):
    return pl.pallas_call(
        _proj_body,
        grid=(5,),
        in_specs=[pl.BlockSpec((2000, DF), lambda i: (i, 0)),
                  pl.BlockSpec((DF, 128), lambda i: (0, 0))],
        out_specs=pl.BlockSpec((2000, 128), lambda i: (i, 0)),
        out_shape=jax.ShapeDtypeStruct((N, 128), jnp.float32),
    )(nf, wpad)


G = 8
GROUP = G * CHUNK
NG = PW // GROUP


def _sc_gather_body(p_hbm, is_hbm, id_hbm, gs_hbm, gd_hbm,
                    iv, rv, sems, semd):
    wid = lax.axis_index("s") * 2 + lax.axis_index("c")
    base = wid * PW
    brow = wid * (PW // CHUNK)

    def load_idx(g, sl):
        pltpu.sync_copy(is_hbm.at[pl.ds(brow + g * G, G)], iv.at[sl, 0])
        pltpu.sync_copy(id_hbm.at[pl.ds(brow + g * G, G)], iv.at[sl, 1])

    def fire(sl):
        for j in range(G):
            pltpu.async_copy(p_hbm.at[iv.at[sl, 0, j]],
                             rv.at[sl, 0, pl.ds(j * CHUNK, CHUNK)], sems)
            pltpu.async_copy(p_hbm.at[iv.at[sl, 1, j]],
                             rv.at[sl, 1, pl.ds(j * CHUNK, CHUNK)], semd)

    def wait(sl):
        for j in range(G):
            pltpu.make_async_copy(p_hbm.at[iv.at[sl, 0, j]],
                                  rv.at[sl, 0, pl.ds(j * CHUNK, CHUNK)],
                                  sems).wait()
            pltpu.make_async_copy(p_hbm.at[iv.at[sl, 1, j]],
                                  rv.at[sl, 1, pl.ds(j * CHUNK, CHUNK)],
                                  semd).wait()

    def store(g, sl):
        off = base + g * GROUP
        pltpu.sync_copy(rv.at[sl, 0], gs_hbm.at[pl.ds(off, GROUP)])
        pltpu.sync_copy(rv.at[sl, 1], gd_hbm.at[pl.ds(off, GROUP)])

    load_idx(0, 0)
    fire(0)

    @pl.loop(0, NG, step=2)
    def _(g):
        load_idx(g + 1, 1)
        fire(1)
        wait(0)

        @pl.when(g + 2 < NG)
        def _():
            load_idx(g + 2, 0)

        store(g, 0)

        @pl.when(g + 2 < NG)
        def _():
            fire(0)

        wait(1)
        store(g + 1, 1)


def _sc_gather(p8, idx_s2, idx_d2):
    mesh = plsc.VectorSubcoreMesh(core_axis_name="c", subcore_axis_name="s")
    f = pl.kernel(
        _sc_gather_body,
        mesh=mesh,
        compiler_params=pltpu.CompilerParams(use_tc_tiling_on_sc=False),
        out_type=[jax.ShapeDtypeStruct((EP, DE), jnp.float32),
                  jax.ShapeDtypeStruct((EP, DE), jnp.float32)],
        scratch_types=[pltpu.VMEM((2, 2, G, CHUNK), jnp.int32),
                       pltpu.VMEM((2, 2, GROUP, DE), jnp.float32),
                       pltpu.SemaphoreType.DMA,
                       pltpu.SemaphoreType.DMA],
    )
    return f(p8, idx_s2, idx_d2)


MB = 4000


def _msg_body(ef_ref, et_ref, gs_ref, gd_ref, we_ref, be_ref, o_ref):
    x = ef_ref[...]
    et = et_ref[...]
    acc = gs_ref[...] + gd_ref[...]
    for t in range(TE):
        y = _dot(x, we_ref[t]) + be_ref[t]
        acc += jnp.where(et == t, y, 0.0)
    o_ref[...] = jnp.maximum(acc, 0.0)


def _msg(ef, et2, gs, gd, we_e, be_pad):
    return pl.pallas_call(
        _msg_body,
        grid=(E // MB,),
        in_specs=[pl.BlockSpec((MB, DE), lambda i: (i, 0)),
                  pl.BlockSpec((MB, 1), lambda i: (i, 0)),
                  pl.BlockSpec((MB, DE), lambda i: (i, 0)),
                  pl.BlockSpec((MB, DE), lambda i: (i, 0)),
                  pl.BlockSpec((TE, DE, DE), lambda i: (0, 0, 0)),
                  pl.BlockSpec((8, DE), lambda i: (0, 0))],
        out_specs=pl.BlockSpec((MB, DE), lambda i: (i, 0)),
        out_shape=jax.ShapeDtypeStruct((E, DE), jnp.float32),
    )(ef, et2, gs, gd, we_e, be_pad)


PWS = E // NW
NFULL = PWS // CHUNK
TAIL = PWS - NFULL * CHUNK


def _sc_scatter_body(val_hbm, didx_hbm, zero_hbm, part_hbm,
                     vv, iv, vt, it, agg_sh, sem):
    cid = lax.axis_index("c")
    sid = lax.axis_index("s")

    @pl.when(sid == 0)
    def _():
        pltpu.sync_copy(zero_hbm, agg_sh)

    plsc.subcore_barrier()

    wid = sid * 2 + cid
    base = wid * PWS

    @pl.loop(0, NFULL)
    def _(c):
        off = base + c * CHUNK
        pltpu.sync_copy(val_hbm.at[pl.ds(off, CHUNK)], vv)
        pltpu.sync_copy(didx_hbm.at[pl.ds(off, CHUNK)], iv)
        pltpu.sync_copy(vv, agg_sh.at[iv], add=True)

    off = base + NFULL * CHUNK
    pltpu.sync_copy(val_hbm.at[pl.ds(off, TAIL)], vt)
    pltpu.sync_copy(didx_hbm.at[pl.ds(off, TAIL)], it)
    pltpu.sync_copy(vt, agg_sh.at[it], add=True)

    plsc.subcore_barrier()
    rows = N // 16
    pltpu.sync_copy(agg_sh.at[pl.ds(sid * rows, rows)],
                    part_hbm.at[cid, pl.ds(sid * rows, rows)])


def _sc_scatter(vals, dst_idx, zeros_n):
    mesh = plsc.VectorSubcoreMesh(core_axis_name="c", subcore_axis_name="s")
    f = pl.kernel(
        _sc_scatter_body,
        mesh=mesh,
        compiler_params=pltpu.CompilerParams(use_tc_tiling_on_sc=False),
        out_type=jax.ShapeDtypeStruct((2, N, DE), jnp.float32),
        scratch_types=[pltpu.VMEM((CHUNK, DE), jnp.float32),
                       pltpu.VMEM((CHUNK,), jnp.int32),
                       pltpu.VMEM((TAIL, DE), jnp.float32),
                       pltpu.VMEM((TAIL,), jnp.int32),
                       pltpu.VMEM_SHARED((N, DE), jnp.float32),
                       pltpu.SemaphoreType.DMA],
    )
    return f(vals, dst_idx, zeros_n)


def _node_body(part_ref, nf_ref, nt_ref, wa_ref, wb_ref, bn_ref, o_ref):
    agg = part_ref[0] + part_ref[1]
    x = nf_ref[...]
    nt = nt_ref[...]
    y0 = jnp.maximum(_dot(agg, wa_ref[0]) + _dot(x, wb_ref[0]) + bn_ref[0], 0.0)
    y1 = jnp.maximum(_dot(agg, wa_ref[1]) + _dot(x, wb_ref[1]) + bn_ref[1], 0.0)
    o_ref[...] = jnp.where(nt == 0, y0, y1)


def _node(part, nf, ntype2, wa, wb, bn_pad):
    return pl.pallas_call(
        _node_body,
        grid=(5,),
        in_specs=[pl.BlockSpec((2, 2000, DE), lambda i: (0, i, 0)),
                  pl.BlockSpec((2000, DF), lambda i: (i, 0)),
                  pl.BlockSpec((2000, 1), lambda i: (i, 0)),
                  pl.BlockSpec((TN, DE, DF), lambda i: (0, 0, 0)),
                  pl.BlockSpec((TN, DF, DF), lambda i: (0, 0, 0)),
                  pl.BlockSpec((8, DF), lambda i: (0, 0))],
        out_specs=pl.BlockSpec((2000, DF), lambda i: (i, 0)),
        out_shape=jax.ShapeDtypeStruct((N, DF), jnp.float32),
    )(part, nf, ntype2, wa, wb, bn_pad)


def kernel(nf, ef, edge_index, etype, ntype, We, be, Wn, bn):
    src = edge_index[0]
    dst = edge_index[1]

    ws = jnp.transpose(We[:, DE:DE + DF, :], (1, 0, 2)).reshape(DF, TE * DE)
    wd = jnp.transpose(We[:, DE + DF:, :], (1, 0, 2)).reshape(DF, TE * DE)
    wpad = jnp.concatenate(
        [ws, wd, jnp.zeros((DF, 128 - 2 * TE * DE), jnp.float32)], axis=1)
    we_e = We[:, :DE, :]
    be_pad = jnp.concatenate(
        [be, jnp.zeros((8 - TE, DE), jnp.float32)], axis=0)
    wa = Wn[:, :DE, :]
    wb = Wn[:, DE:, :]
    bn_pad = jnp.concatenate(
        [bn, jnp.zeros((8 - TN, DF), jnp.float32)], axis=0)

    pad = EP - E
    idx_s2 = jnp.pad(src * 8 + etype, (0, pad)).reshape(EP // CHUNK, CHUNK)
    idx_d2 = jnp.pad(dst * 8 + 3 + etype, (0, pad)).reshape(EP // CHUNK, CHUNK)
    et2 = etype.reshape(E, 1)
    ntype2 = ntype.reshape(N, 1)
    zeros_n = jnp.zeros((N, DE), jnp.float32)

    p = _proj(nf, wpad)
    p8 = p.reshape(N * 8, DE)

    gs, gd = _sc_gather(p8, idx_s2, idx_d2)

    msg = _msg(ef, et2, gs, gd, we_e, be_pad)

    part = _sc_scatter(msg, dst, zeros_n)

    updated_nf = _node(part, nf, ntype2, wa, wb, bn_pad)

    return (updated_nf, msg)


# speedup vs baseline: 2.1495x; 2.1495x over previous
import functools

import jax
import jax.numpy as jnp
from jax import lax
from jax.experimental import pallas as pl
from jax.experimental.pallas import tpu as pltpu
from jax.experimental.pallas import tpu_sc as plsc

N = 10000
E = 320000
DF = 128
DE = 16
TE = 3
TN = 2

NW = 32
CHUNK = 128
NCHUNK = 80
PW = CHUNK * NCHUNK
EP = NW * PW
ROWS = EP * DE // 128
ROWS_REAL = E * DE // 128

_HIGH = lax.Precision.HIGHEST


def _dot(a, b):
    return lax.dot_general(a, b, (((1,), (0,)), ((), ())),
                           precision=_HIGH, preferred_element_type=jnp.float32)


def _proj_body(nf_ref, w_ref, b_ref, o_ref):
    o_ref[...] = _dot(nf_ref[...], w_ref[...]) + b_ref[0:1, :]


def _proj(nf, wpad, bpad):
    return pl.pallas_call(
        _proj_body,
        grid=(5,),
        in_specs=[pl.BlockSpec((2000, DF), lambda i: (i, 0)),
                  pl.BlockSpec((DF, 128), lambda i: (0, 0)),
                  pl.BlockSpec((8, 128), lambda i: (0, 0))],
        out_specs=pl.BlockSpec((2000, 128), lambda i: (i, 0)),
        out_shape=jax.ShapeDtypeStruct((N, 128), jnp.float32),
    )(nf, wpad, bpad)


G = 8
GROUP = G * CHUNK
NG = PW // GROUP


def _sc_gather_body(p_hbm, is_hbm, id_hbm, gs_hbm, gd_hbm,
                    iv, rv, sems, semd):
    wid = lax.axis_index("s") * 2 + lax.axis_index("c")
    base = wid * PW
    brow = wid * (PW // CHUNK)

    def load_idx(g, sl):
        pltpu.sync_copy(is_hbm.at[pl.ds(brow + g * G, G)], iv.at[sl, 0])
        pltpu.sync_copy(id_hbm.at[pl.ds(brow + g * G, G)], iv.at[sl, 1])

    def fire(sl):
        for j in range(G):
            pltpu.async_copy(p_hbm.at[iv.at[sl, 0, j]],
                             rv.at[sl, 0, pl.ds(j * CHUNK, CHUNK)], sems)
            pltpu.async_copy(p_hbm.at[iv.at[sl, 1, j]],
                             rv.at[sl, 1, pl.ds(j * CHUNK, CHUNK)], semd)

    def wait(sl):
        for j in range(G):
            pltpu.make_async_copy(p_hbm.at[iv.at[sl, 0, j]],
                                  rv.at[sl, 0, pl.ds(j * CHUNK, CHUNK)],
                                  sems).wait()
            pltpu.make_async_copy(p_hbm.at[iv.at[sl, 1, j]],
                                  rv.at[sl, 1, pl.ds(j * CHUNK, CHUNK)],
                                  semd).wait()

    def store(g, sl):
        off = base + g * GROUP
        pltpu.sync_copy(rv.at[sl, 0], gs_hbm.at[pl.ds(off, GROUP)])
        pltpu.sync_copy(rv.at[sl, 1], gd_hbm.at[pl.ds(off, GROUP)])

    load_idx(0, 0)
    fire(0)

    @pl.loop(0, NG, step=2)
    def _(g):
        load_idx(g + 1, 1)
        fire(1)
        wait(0)

        @pl.when(g + 2 < NG)
        def _():
            load_idx(g + 2, 0)

        store(g, 0)

        @pl.when(g + 2 < NG)
        def _():
            fire(0)

        wait(1)
        store(g + 1, 1)


def _sc_gather(p8, idx_s2, idx_d2):
    mesh = plsc.VectorSubcoreMesh(core_axis_name="c", subcore_axis_name="s")
    f = pl.kernel(
        _sc_gather_body,
        mesh=mesh,
        compiler_params=pltpu.CompilerParams(use_tc_tiling_on_sc=False),
        out_type=[jax.ShapeDtypeStruct((EP, DE), jnp.float32),
                  jax.ShapeDtypeStruct((EP, DE), jnp.float32)],
        scratch_types=[pltpu.VMEM((2, 2, G, CHUNK), jnp.int32),
                       pltpu.VMEM((2, 2, GROUP, DE), jnp.float32),
                       pltpu.SemaphoreType.DMA,
                       pltpu.SemaphoreType.DMA],
    )
    return f(p8, idx_s2, idx_d2)


MB = 800


def _msg_body(ef_ref, et_ref, gs_ref, gd_ref, wbd_ref, o_ref):
    x = ef_ref[...]
    et = et_ref[...]
    acc = gs_ref[...] + gd_ref[...]
    for t in range(TE):
        acc += jnp.where(et == t, _dot(x, wbd_ref[t]), 0.0)
    o_ref[...] = jnp.maximum(acc, 0.0)


def _msg(ef_rs, et_rep, gs_rs, gd_rs, wbd):
    return pl.pallas_call(
        _msg_body,
        grid=(ROWS_REAL // MB,),
        in_specs=[pl.BlockSpec((MB, 128), lambda i: (i, 0)),
                  pl.BlockSpec((MB, 128), lambda i: (i, 0)),
                  pl.BlockSpec((MB, 128), lambda i: (i, 0)),
                  pl.BlockSpec((MB, 128), lambda i: (i, 0)),
                  pl.BlockSpec((TE, 128, 128), lambda i: (0, 0, 0))],
        out_specs=pl.BlockSpec((MB, 128), lambda i: (i, 0)),
        out_shape=jax.ShapeDtypeStruct((ROWS_REAL, 128), jnp.float32),
    )(ef_rs, et_rep, gs_rs, gd_rs, wbd)


PWS = E // NW
NFULL = PWS // CHUNK
TAIL = PWS - NFULL * CHUNK


def _sc_scatter_body(val_hbm, didx_hbm, zero_hbm, part_hbm,
                     vv, iv, vt, it, agg_sh, sem):
    cid = lax.axis_index("c")
    sid = lax.axis_index("s")

    @pl.when(sid == 0)
    def _():
        pltpu.sync_copy(zero_hbm, agg_sh)

    plsc.subcore_barrier()

    wid = sid * 2 + cid
    base = wid * PWS

    @pl.loop(0, NFULL)
    def _(c):
        off = base + c * CHUNK
        pltpu.sync_copy(val_hbm.at[pl.ds(off, CHUNK)], vv)
        pltpu.sync_copy(didx_hbm.at[pl.ds(off, CHUNK)], iv)
        pltpu.sync_copy(vv, agg_sh.at[iv], add=True)

    off = base + NFULL * CHUNK
    pltpu.sync_copy(val_hbm.at[pl.ds(off, TAIL)], vt)
    pltpu.sync_copy(didx_hbm.at[pl.ds(off, TAIL)], it)
    pltpu.sync_copy(vt, agg_sh.at[it], add=True)

    plsc.subcore_barrier()
    rows = N // 16
    pltpu.sync_copy(agg_sh.at[pl.ds(sid * rows, rows)],
                    part_hbm.at[cid, pl.ds(sid * rows, rows)])


def _sc_scatter(vals, dst_idx, zeros_n):
    mesh = plsc.VectorSubcoreMesh(core_axis_name="c", subcore_axis_name="s")
    f = pl.kernel(
        _sc_scatter_body,
        mesh=mesh,
        compiler_params=pltpu.CompilerParams(use_tc_tiling_on_sc=False),
        out_type=jax.ShapeDtypeStruct((2, N, DE), jnp.float32),
        scratch_types=[pltpu.VMEM((CHUNK, DE), jnp.float32),
                       pltpu.VMEM((CHUNK,), jnp.int32),
                       pltpu.VMEM((TAIL, DE), jnp.float32),
                       pltpu.VMEM((TAIL,), jnp.int32),
                       pltpu.VMEM_SHARED((N, DE), jnp.float32),
                       pltpu.SemaphoreType.DMA],
    )
    return f(vals, dst_idx, zeros_n)


def _node_body(part_ref, nf_ref, nt_ref, wa_ref, wb_ref, bn_ref, o_ref):
    agg = part_ref[0] + part_ref[1]
    x = nf_ref[...]
    nt = nt_ref[...]
    y0 = jnp.maximum(_dot(agg, wa_ref[0]) + _dot(x, wb_ref[0]) + bn_ref[0], 0.0)
    y1 = jnp.maximum(_dot(agg, wa_ref[1]) + _dot(x, wb_ref[1]) + bn_ref[1], 0.0)
    o_ref[...] = jnp.where(nt == 0, y0, y1)


def _node(part, nf, ntype2, wa, wb, bn_pad):
    return pl.pallas_call(
        _node_body,
        grid=(5,),
        in_specs=[pl.BlockSpec((2, 2000, DE), lambda i: (0, i, 0)),
                  pl.BlockSpec((2000, DF), lambda i: (i, 0)),
                  pl.BlockSpec((2000, 1), lambda i: (i, 0)),
                  pl.BlockSpec((TN, DE, DF), lambda i: (0, 0, 0)),
                  pl.BlockSpec((TN, DF, DF), lambda i: (0, 0, 0)),
                  pl.BlockSpec((8, DF), lambda i: (0, 0))],
        out_specs=pl.BlockSpec((2000, DF), lambda i: (i, 0)),
        out_shape=jax.ShapeDtypeStruct((N, DF), jnp.float32),
    )(part, nf, ntype2, wa, wb, bn_pad)


def kernel(nf, ef, edge_index, etype, ntype, We, be, Wn, bn):
    src = edge_index[0]
    dst = edge_index[1]

    ws = jnp.transpose(We[:, DE:DE + DF, :], (1, 0, 2)).reshape(DF, TE * DE)
    wd = jnp.transpose(We[:, DE + DF:, :], (1, 0, 2)).reshape(DF, TE * DE)
    wpad = jnp.concatenate(
        [ws, wd, jnp.zeros((DF, 128 - 2 * TE * DE), jnp.float32)], axis=1)
    eye8 = jnp.eye(8, dtype=jnp.float32)
    wbd = jax.vmap(lambda w: jnp.kron(eye8, w))(We[:, :DE, :])
    bpad = jnp.tile(jnp.concatenate(
        [jnp.zeros((3, DE), jnp.float32), be,
         jnp.zeros((2, DE), jnp.float32)], axis=0).reshape(1, 128), (8, 1))
    wa = Wn[:, :DE, :]
    wb = Wn[:, DE:, :]
    bn_pad = jnp.concatenate(
        [bn, jnp.zeros((8 - TN, DF), jnp.float32)], axis=0)

    pad = EP - E
    idx_s2 = jnp.pad(src * 8 + etype, (0, pad)).reshape(EP // CHUNK, CHUNK)
    idx_d2 = jnp.pad(dst * 8 + 3 + etype, (0, pad)).reshape(EP // CHUNK, CHUNK)
    ef_rs = ef.reshape(ROWS_REAL, 128)
    et_rep = jnp.repeat(etype, DE).reshape(ROWS_REAL, 128)
    ntype2 = ntype.reshape(N, 1)
    zeros_n = jnp.zeros((N, DE), jnp.float32)

    p = _proj(nf, wpad, bpad)
    p8 = p.reshape(N * 8, DE)

    gs, gd = _sc_gather(p8, idx_s2, idx_d2)
    gs_rs = gs.reshape(ROWS, 128)
    gd_rs = gd.reshape(ROWS, 128)

    msg = _msg(ef_rs, et_rep, gs_rs, gd_rs, wbd).reshape(E, DE)

    part = _sc_scatter(msg, dst, zeros_n)

    updated_nf = _node(part, nf, ntype2, wa, wb, bn_pad)

    return (updated_nf, msg)


# speedup vs baseline: 2.5568x; 1.1895x over previous
import functools

import jax
import jax.numpy as jnp
from jax import lax
from jax.experimental import pallas as pl
from jax.experimental.pallas import tpu as pltpu
from jax.experimental.pallas import tpu_sc as plsc

N = 10000
E = 320000
DF = 128
DE = 16
TE = 3
TN = 2

NW = 32
CHUNK = 128
NCHUNK = 80
PW = CHUNK * NCHUNK
EP = NW * PW
ROWS = EP * DE // 128
ROWS_REAL = E * DE // 128

_HIGH = lax.Precision.HIGHEST


def _dot(a, b):
    return lax.dot_general(a, b, (((1,), (0,)), ((), ())),
                           precision=_HIGH, preferred_element_type=jnp.float32)


def _proj_body(nf_ref, w_ref, b_ref, o_ref):
    o_ref[...] = _dot(nf_ref[...], w_ref[...]) + b_ref[0:1, :]


def _proj(nf, wpad, bpad):
    return pl.pallas_call(
        _proj_body,
        grid=(5,),
        in_specs=[pl.BlockSpec((2000, DF), lambda i: (i, 0)),
                  pl.BlockSpec((DF, 96), lambda i: (0, 0)),
                  pl.BlockSpec((8, 96), lambda i: (0, 0))],
        out_specs=pl.BlockSpec((2000, 96), lambda i: (i, 0)),
        out_shape=jax.ShapeDtypeStruct((N, 96), jnp.float32),
    )(nf, wpad, bpad)


G = 8
GROUP = G * CHUNK
NG = PW // GROUP


def _sc_gather_body(p_hbm, is_hbm, id_hbm, gs_hbm, gd_hbm,
                    iv, rv, p_sh, sems, semd):
    cid = lax.axis_index("c")
    sid = lax.axis_index("s")

    @pl.when(sid == 0)
    def _():
        pltpu.sync_copy(p_hbm, p_sh)

    plsc.subcore_barrier()

    wid = sid * 2 + cid
    base = wid * PW
    brow = wid * (PW // CHUNK)

    def load_idx(g, sl):
        pltpu.sync_copy(is_hbm.at[pl.ds(brow + g * G, G)], iv.at[sl, 0])
        pltpu.sync_copy(id_hbm.at[pl.ds(brow + g * G, G)], iv.at[sl, 1])

    def fire(sl):
        for j in range(G):
            pltpu.async_copy(p_sh.at[iv.at[sl, 0, j]],
                             rv.at[sl, 0, pl.ds(j * CHUNK, CHUNK)], sems)
            pltpu.async_copy(p_sh.at[iv.at[sl, 1, j]],
                             rv.at[sl, 1, pl.ds(j * CHUNK, CHUNK)], semd)

    def wait(sl):
        for j in range(G):
            pltpu.make_async_copy(p_sh.at[iv.at[sl, 0, j]],
                                  rv.at[sl, 0, pl.ds(j * CHUNK, CHUNK)],
                                  sems).wait()
            pltpu.make_async_copy(p_sh.at[iv.at[sl, 1, j]],
                                  rv.at[sl, 1, pl.ds(j * CHUNK, CHUNK)],
                                  semd).wait()

    def store(g, sl):
        off = base + g * GROUP
        pltpu.sync_copy(rv.at[sl, 0], gs_hbm.at[pl.ds(off, GROUP)])
        pltpu.sync_copy(rv.at[sl, 1], gd_hbm.at[pl.ds(off, GROUP)])

    load_idx(0, 0)
    fire(0)

    @pl.loop(0, NG, step=2)
    def _(g):
        load_idx(g + 1, 1)
        fire(1)
        wait(0)

        @pl.when(g + 2 < NG)
        def _():
            load_idx(g + 2, 0)

        store(g, 0)

        @pl.when(g + 2 < NG)
        def _():
            fire(0)

        wait(1)
        store(g + 1, 1)


def _sc_gather(p8, idx_s2, idx_d2):
    mesh = plsc.VectorSubcoreMesh(core_axis_name="c", subcore_axis_name="s")
    f = pl.kernel(
        _sc_gather_body,
        mesh=mesh,
        compiler_params=pltpu.CompilerParams(use_tc_tiling_on_sc=False),
        out_type=[jax.ShapeDtypeStruct((EP, DE), jnp.float32),
                  jax.ShapeDtypeStruct((EP, DE), jnp.float32)],
        scratch_types=[pltpu.VMEM((2, 2, G, CHUNK), jnp.int32),
                       pltpu.VMEM((2, 2, GROUP, DE), jnp.float32),
                       pltpu.VMEM_SHARED((N * 6, DE), jnp.float32),
                       pltpu.SemaphoreType.DMA,
                       pltpu.SemaphoreType.DMA],
    )
    return f(p8, idx_s2, idx_d2)


MB = 800


def _msg_body(ef_ref, et_ref, gs_ref, gd_ref, wbd_ref, o_ref):
    x = ef_ref[...]
    et = et_ref[...]
    acc = gs_ref[...] + gd_ref[...]
    for t in range(TE):
        acc += jnp.where(et == t, _dot(x, wbd_ref[t]), 0.0)
    o_ref[...] = jnp.maximum(acc, 0.0)


def _msg(ef_rs, et_rep, gs_rs, gd_rs, wbd):
    return pl.pallas_call(
        _msg_body,
        grid=(ROWS_REAL // MB,),
        in_specs=[pl.BlockSpec((MB, 128), lambda i: (i, 0)),
                  pl.BlockSpec((MB, 128), lambda i: (i, 0)),
                  pl.BlockSpec((MB, 128), lambda i: (i, 0)),
                  pl.BlockSpec((MB, 128), lambda i: (i, 0)),
                  pl.BlockSpec((TE, 128, 128), lambda i: (0, 0, 0))],
        out_specs=pl.BlockSpec((MB, 128), lambda i: (i, 0)),
        out_shape=jax.ShapeDtypeStruct((ROWS_REAL, 128), jnp.float32),
    )(ef_rs, et_rep, gs_rs, gd_rs, wbd)


NCH = E // CHUNK
KMAX = (NCH + NW - 1) // NW


def _sc_scatter_body(val_hbm, didx_hbm, zero_hbm, part_hbm,
                     vv, iv, agg_sh, seml0, seml1):
    cid = lax.axis_index("c")
    sid = lax.axis_index("s")

    @pl.when(sid == 0)
    def _():
        pltpu.sync_copy(zero_hbm, agg_sh)

    plsc.subcore_barrier()

    wid = sid * 2 + cid
    sems = (seml0, seml1)

    def aload(k, sl):
        off = (wid + k * NW) * CHUNK
        pltpu.async_copy(val_hbm.at[pl.ds(off, CHUNK)], vv.at[sl], sems[sl])
        pltpu.async_copy(didx_hbm.at[pl.ds(off, CHUNK)], iv.at[sl], sems[sl])

    def wload(k, sl):
        off = (wid + k * NW) * CHUNK
        pltpu.make_async_copy(val_hbm.at[pl.ds(off, CHUNK)], vv.at[sl],
                              sems[sl]).wait()
        pltpu.make_async_copy(didx_hbm.at[pl.ds(off, CHUNK)], iv.at[sl],
                              sems[sl]).wait()

    def sadd(sl):
        pltpu.sync_copy(vv.at[sl], agg_sh.at[iv.at[sl]], add=True)

    nk = 78 + jnp.where(wid + 78 * NW < NCH, 1, 0)

    aload(0, 0)

    @pl.loop(0, KMAX, step=2)
    def _(k):
        @pl.when(k < nk)
        def _():
            @pl.when(k + 1 < nk)
            def _():
                aload(k + 1, 1)

            wload(k, 0)
            sadd(0)

            @pl.when(k + 2 < nk)
            def _():
                aload(k + 2, 0)

            @pl.when(k + 1 < nk)
            def _():
                wload(k + 1, 1)
                sadd(1)

    plsc.subcore_barrier()
    rows = N // 16
    pltpu.sync_copy(agg_sh.at[pl.ds(sid * rows, rows)],
                    part_hbm.at[cid, pl.ds(sid * rows, rows)])


def _sc_scatter(vals, dst_idx, zeros_n):
    mesh = plsc.VectorSubcoreMesh(core_axis_name="c", subcore_axis_name="s")
    f = pl.kernel(
        _sc_scatter_body,
        mesh=mesh,
        compiler_params=pltpu.CompilerParams(use_tc_tiling_on_sc=False),
        out_type=jax.ShapeDtypeStruct((2, N, DE), jnp.float32),
        scratch_types=[pltpu.VMEM((2, CHUNK, DE), jnp.float32),
                       pltpu.VMEM((2, CHUNK), jnp.int32),
                       pltpu.VMEM_SHARED((N, DE), jnp.float32),
                       pltpu.SemaphoreType.DMA,
                       pltpu.SemaphoreType.DMA],
    )
    return f(vals, dst_idx, zeros_n)


def _node_body(part_ref, nf_ref, nt_ref, wa_ref, wb_ref, bn_ref, o_ref):
    agg = part_ref[0] + part_ref[1]
    x = nf_ref[...]
    nt = nt_ref[...]
    y0 = jnp.maximum(_dot(agg, wa_ref[0]) + _dot(x, wb_ref[0]) + bn_ref[0], 0.0)
    y1 = jnp.maximum(_dot(agg, wa_ref[1]) + _dot(x, wb_ref[1]) + bn_ref[1], 0.0)
    o_ref[...] = jnp.where(nt == 0, y0, y1)


def _node(part, nf, ntype2, wa, wb, bn_pad):
    return pl.pallas_call(
        _node_body,
        grid=(5,),
        in_specs=[pl.BlockSpec((2, 2000, DE), lambda i: (0, i, 0)),
                  pl.BlockSpec((2000, DF), lambda i: (i, 0)),
                  pl.BlockSpec((2000, 1), lambda i: (i, 0)),
                  pl.BlockSpec((TN, DE, DF), lambda i: (0, 0, 0)),
                  pl.BlockSpec((TN, DF, DF), lambda i: (0, 0, 0)),
                  pl.BlockSpec((8, DF), lambda i: (0, 0))],
        out_specs=pl.BlockSpec((2000, DF), lambda i: (i, 0)),
        out_shape=jax.ShapeDtypeStruct((N, DF), jnp.float32),
    )(part, nf, ntype2, wa, wb, bn_pad)


def kernel(nf, ef, edge_index, etype, ntype, We, be, Wn, bn):
    src = edge_index[0]
    dst = edge_index[1]

    ws = jnp.transpose(We[:, DE:DE + DF, :], (1, 0, 2)).reshape(DF, TE * DE)
    wd = jnp.transpose(We[:, DE + DF:, :], (1, 0, 2)).reshape(DF, TE * DE)
    wpad = jnp.concatenate([ws, wd], axis=1)
    eye8 = jnp.eye(8, dtype=jnp.float32)
    wbd = jax.vmap(lambda w: jnp.kron(eye8, w))(We[:, :DE, :])
    bpad = jnp.tile(jnp.concatenate(
        [jnp.zeros((3, DE), jnp.float32), be], axis=0).reshape(1, 96), (8, 1))
    wa = Wn[:, :DE, :]
    wb = Wn[:, DE:, :]
    bn_pad = jnp.concatenate(
        [bn, jnp.zeros((8 - TN, DF), jnp.float32)], axis=0)

    pad = EP - E
    idx_s2 = jnp.pad(src * 6 + etype, (0, pad)).reshape(EP // CHUNK, CHUNK)
    idx_d2 = jnp.pad(dst * 6 + 3 + etype, (0, pad)).reshape(EP // CHUNK, CHUNK)
    ef_rs = ef.reshape(ROWS_REAL, 128)
    et_rep = jnp.repeat(etype, DE).reshape(ROWS_REAL, 128)
    ntype2 = ntype.reshape(N, 1)
    zeros_n = jnp.zeros((N, DE), jnp.float32)

    p = _proj(nf, wpad, bpad)
    p8 = p.reshape(N * 6, DE)

    gs, gd = _sc_gather(p8, idx_s2, idx_d2)
    gs_rs = gs.reshape(ROWS, 128)
    gd_rs = gd.reshape(ROWS, 128)

    msg = _msg(ef_rs, et_rep, gs_rs, gd_rs, wbd).reshape(E, DE)

    part = _sc_scatter(msg, dst, zeros_n)

    updated_nf = _node(part, nf, ntype2, wa, wb, bn_pad)

    return (updated_nf, msg)
